# baseline (speedup 1.0000x reference)
import functools

import jax
import jax.numpy as jnp
from jax import lax
from jax.experimental import pallas as pl
from jax.experimental.pallas import tpu as pltpu
from jax.experimental.pallas import tpu_sc as plsc

F32 = jnp.float32
I32 = jnp.int32


def _ru(x, m):
    return (x + m - 1) // m * m


def _mm_body(x_ref, w_ref, b_ref, o_ref, *, relu):
    y = jnp.dot(x_ref[...], w_ref[...], preferred_element_type=F32) + b_ref[...]
    if relu:
        y = jnp.maximum(y, 0.0)
    o_ref[...] = y


def _row_tile(n):
    for t in (2048, 1024, 1000, 512, 500, 256, 200, 128, 8):
        if n % t == 0:
            return t
    return n


def _mm(x, w, b, relu):
    n, k = x.shape
    m = w.shape[1]
    tr = _row_tile(n)
    return pl.pallas_call(
        functools.partial(_mm_body, relu=relu),
        grid=(n // tr,),
        in_specs=[
            pl.BlockSpec((tr, k), lambda i: (i, 0)),
            pl.BlockSpec((k, m), lambda i: (0, 0)),
            pl.BlockSpec((1, m), lambda i: (0, 0)),
        ],
        out_specs=pl.BlockSpec((tr, m), lambda i: (i, 0)),
        out_shape=jax.ShapeDtypeStruct((n, m), F32),
    )(x, w, b)


def _nodeupd_body(x_ref, nb_ref, w1_ref, w2_ref, b_ref, o_ref):
    nb = nb_ref[0] + nb_ref[1]
    y = (
        jnp.dot(x_ref[...], w1_ref[...], preferred_element_type=F32)
        + jnp.dot(nb, w2_ref[...], preferred_element_type=F32)
        + b_ref[...]
    )
    o_ref[...] = jnp.maximum(y, 0.0)


def _nodeupd(x, nbr, w1, w2, b):
    n, k = x.shape
    m = w1.shape[1]
    tr = _row_tile(n)
    return pl.pallas_call(
        _nodeupd_body,
        grid=(n // tr,),
        in_specs=[
            pl.BlockSpec((tr, k), lambda i: (i, 0)),
            pl.BlockSpec((2, tr, k), lambda i: (0, i, 0)),
            pl.BlockSpec((k, m), lambda i: (0, 0)),
            pl.BlockSpec((k, m), lambda i: (0, 0)),
            pl.BlockSpec((1, m), lambda i: (0, 0)),
        ],
        out_specs=pl.BlockSpec((tr, m), lambda i: (i, 0)),
        out_shape=jax.ShapeDtypeStruct((n, m), F32),
    )(x, nbr, w1, w2, b)


def _remap_body(d_ref, o_ref, *, vb):
    k = pl.program_id(0)
    lo = k * vb
    d = d_ref[...]
    m = (d >= lo) & (d < lo + vb)
    o_ref[...] = jnp.where(m, d - lo, vb)


def _remap(dst2, nb, vb):
    r = dst2.shape[0]
    tr = _row_tile(r)
    return pl.pallas_call(
        functools.partial(_remap_body, vb=vb),
        grid=(nb, r // tr),
        in_specs=[pl.BlockSpec((tr, 128), lambda k, i: (i, 0))],
        out_specs=pl.BlockSpec((tr, 128), lambda k, i, rt=r // tr: (k * rt + i, 0)),
        out_shape=jax.ShapeDtypeStruct((nb * r, 128), I32),
    )(dst2)


def _diff_body(p_ref, r_ref, o_ref):
    o_ref[...] = p_ref[...] - r_ref[...]


def _diff(p_h, r_h, nc):
    v, f = r_h.shape
    tr = _row_tile(v)
    return pl.pallas_call(
        _diff_body,
        grid=(nc, v // tr),
        in_specs=[
            pl.BlockSpec((tr, f), lambda c, i, vt=v // tr: (c * vt + i, 0)),
            pl.BlockSpec((tr, f), lambda c, i: (i, 0)),
        ],
        out_specs=pl.BlockSpec((tr, f), lambda c, i, vt=v // tr: (c * vt + i, 0)),
        out_shape=jax.ShapeDtypeStruct((nc * v, f), F32),
    )(p_h, r_h)


def _pool_body(x_ref, o_ref):
    part = jnp.sum(x_ref[...], axis=0, keepdims=True)
    c = pl.program_id(0)

    @pl.when((c == 0) & (pl.program_id(1) == 0))
    def _():
        o_ref[...] = jnp.zeros_like(o_ref)

    rows = lax.broadcasted_iota(I32, (8, 1), 0)
    o_ref[...] = o_ref[...] + jnp.where(rows == c, part, 0.0)


def _pool(x, nc, v):
    f = x.shape[1]
    tr = _row_tile(v)
    out = pl.pallas_call(
        _pool_body,
        grid=(nc, v // tr),
        in_specs=[pl.BlockSpec((tr, f), lambda c, i, vt=v // tr: (c * vt + i, 0))],
        out_specs=pl.BlockSpec((8, f), lambda c, i: (0, 0)),
        out_shape=jax.ShapeDtypeStruct((8, f), F32),
    )(x)
    return out[:nc]


def _mlp_body(g_ref, w1_ref, b1_ref, w2_ref, b2_ref, sc_ref, o_ref):
    h = jnp.maximum(
        jnp.dot(g_ref[...], w1_ref[...], preferred_element_type=F32) + b1_ref[...],
        0.0,
    )
    o_ref[...] = (
        jnp.sum(h * w2_ref[...], axis=1, keepdims=True) + b2_ref[...] + sc_ref[...]
    )


def _mlp(g, w1, b1, w2row, b2, scores):
    nc = g.shape[0]
    return pl.pallas_call(
        _mlp_body,
        out_shape=jax.ShapeDtypeStruct((nc, 1), F32),
    )(g, w1, b1, w2row, b2, scores)


_NSUB = 16
_BLK = 128
_MAXACC = 11776


def _graph_cfg(v, e):
    ep = _ru(e, 32 * 1024)
    nb = -(-v // _MAXACC)
    vb = _ru(-(-v // nb), _BLK)
    capt = ep // 32
    return ep, nb, vb, capt


@functools.lru_cache(maxsize=None)
def _make_edge_op(v, e):
    ep, nb, vb, capt = _graph_cfg(v, e)
    ra = vb + _BLK
    vo = nb * vb
    cpr = capt // _BLK
    rr = ep // _BLK
    ngrp = capt // (8 * _BLK)
    zrows = ra // _NSUB
    rps = vb // _NSUB
    mesh = plsc.VectorSubcoreMesh(core_axis_name="c", subcore_axis_name="s")

    @functools.partial(
        pl.kernel,
        out_type=jax.ShapeDtypeStruct((2, vo, _BLK), F32),
        mesh=mesh,
        scratch_types=[
            pltpu.VMEM_SHARED((ra, _BLK), F32),
            pltpu.VMEM((8, _BLK), I32),
            pltpu.VMEM((8, _BLK), I32),
            pltpu.VMEM((_BLK, _BLK), F32),
            pltpu.VMEM((_BLK, _BLK), F32),
            pltpu.SemaphoreType.DMA,
            pltpu.SemaphoreType.DMA,
        ],
    )
    def edge_op(p_hbm, q_hbm, src2, didx_all, out, acc, sidx, didx, qbuf, zbuf,
                gsem, ssem):
        c = lax.axis_index("c")
        s = lax.axis_index("s")
        t = c * _NSUB + s

        @pl.loop(0, _BLK)
        def _(r):
            for kk in range(8):
                zbuf[r, pl.ds(kk * 16, 16)] = jnp.zeros((16,), F32)

        for k in range(nb):
            zb0 = pl.multiple_of(s * zrows, 8)
            off = 0
            while off < zrows:
                blk = min(_BLK, zrows - off)
                pltpu.sync_copy(zbuf.at[pl.ds(0, blk)], acc.at[pl.ds(zb0 + off, blk)])
                off += blk
            plsc.subcore_barrier()

            @pl.loop(0, ngrp)
            def _(g):
                rb = pl.multiple_of(t * cpr + g * 8, 8)
                pltpu.sync_copy(src2.at[pl.ds(rb, 8)], sidx)
                pltpu.sync_copy(didx_all.at[pl.ds(k * rr + rb, 8)], didx)
                for j in range(8):
                    eoff = pl.multiple_of((rb + j) * _BLK, _BLK)
                    pltpu.sync_copy(q_hbm.at[pl.ds(eoff, _BLK)], qbuf)
                    pltpu.async_copy(p_hbm.at[sidx.at[j]], qbuf, gsem, add=True).wait()

                    @pl.loop(0, _BLK)
                    def _(r):
                        for kk in range(8):
                            sl = pl.ds(kk * 16, 16)
                            qbuf[r, sl] = jnp.maximum(qbuf[r, sl], 0.0)

                    pltpu.async_copy(qbuf, acc.at[didx.at[j]], ssem, add=True).wait()

            plsc.subcore_barrier()
            r0 = pl.multiple_of(s * rps, 8)
            pltpu.sync_copy(
                acc.at[pl.ds(r0, rps)],
                out.at[c, pl.ds(pl.multiple_of(k * vb + r0, 8), rps)],
            )
            plsc.subcore_barrier()

    return edge_op


def _pad_graph(edge_index, edge_feats, v, ep):
    e = edge_feats.shape[0]
    src = edge_index[0].astype(I32)
    dst = edge_index[1].astype(I32)
    src = jnp.pad(src, (0, ep - e)).reshape(ep // _BLK, _BLK)
    dst = jnp.pad(dst, (0, ep - e), constant_values=v).reshape(ep // _BLK, _BLK)
    ef = jnp.pad(edge_feats, ((0, ep - e), (0, 0)))
    return src, dst, ef


def kernel(reactant_edge_index, reactant_node_feats, reactant_edge_feats,
           product_edge_index, product_node_feats, product_edge_feats,
           candidate_scores, gnn_Wp, gnn_bp, gnn_Wm, gnn_bm, gnn_Wn, gnn_bn,
           dgnn_Wm, dgnn_bm, dgnn_Wn, dgnn_bn, pred_W1, pred_b1, pred_W2,
           pred_b2):
    v, f = reactant_node_feats.shape
    vp = product_node_feats.shape[0]
    nc = candidate_scores.shape[0]
    er = reactant_edge_feats.shape[0]
    ep_e = product_edge_feats.shape[0]

    epr, nbr_r, vbr, _ = _graph_cfg(v, er)
    epp, nbr_p, vbp, _ = _graph_cfg(vp, ep_e)

    r_src, r_dst, r_ef = _pad_graph(reactant_edge_index, reactant_edge_feats, v, epr)
    p_src, p_dst, p_ef = _pad_graph(product_edge_index, product_edge_feats, vp, epp)

    bp_ = gnn_bp.reshape(1, -1)
    bm_ = gnn_bm.reshape(1, -1)
    bn_ = gnn_bn.reshape(1, -1)
    dbm_ = dgnn_bm.reshape(1, -1)
    dbn_ = dgnn_bn.reshape(1, -1)
    wm_h, wm_e = gnn_Wm[:f], gnn_Wm[f:]
    wn_1, wn_2 = gnn_Wn[:f], gnn_Wn[f:]
    dwm_h, dwm_e = dgnn_Wm[:f], dgnn_Wm[f:]
    dwn_1, dwn_2 = dgnn_Wn[:f], dgnn_Wn[f:]
    zb = jnp.zeros((1, f), F32)

    rdix = _remap(r_dst, nbr_r, vbr)
    pdix = _remap(p_dst, nbr_p, vbp)
    edge_r = _make_edge_op(v, er)
    edge_p = _make_edge_op(vp, ep_e)

    q_r = _mm(r_ef, wm_e, bm_, relu=False)
    q_p = _mm(p_ef, wm_e, bm_, relu=False)
    q_d = _mm(p_ef, dwm_e, dbm_, relu=False)

    r_h = _mm(reactant_node_feats, gnn_Wp, bp_, relu=True)
    p_h = _mm(product_node_feats, gnn_Wp, bp_, relu=True)

    for _ in range(3):
        pr = _mm(r_h, wm_h, zb, relu=False)
        nbr = edge_r(pr, q_r, r_src, rdix)
        r_h = _nodeupd(r_h, nbr, wn_1, wn_2, bn_)

        pp = _mm(p_h, wm_h, zb, relu=False)
        nbrp = edge_p(pp, q_p, p_src, pdix)
        p_h = _nodeupd(p_h, nbrp, wn_1, wn_2, bn_)

    d_h = _diff(p_h, r_h, nc)
    pd = _mm(d_h, dwm_h, zb, relu=False)
    nbrd = edge_p(pd, q_d, p_src, pdix)
    d_h = _nodeupd(d_h, nbrd, dwn_1, dwn_2, dbn_)

    g_feats = _pool(d_h, nc, v)
    return _mlp(g_feats, pred_W1, pred_b1.reshape(1, -1),
                pred_W2.reshape(1, -1), pred_b2.reshape(1, 1), candidate_scores)


# speedup vs baseline: 1.0724x; 1.0724x over previous
import functools

import jax
import jax.numpy as jnp
from jax import lax
from jax.experimental import pallas as pl
from jax.experimental.pallas import tpu as pltpu
from jax.experimental.pallas import tpu_sc as plsc

F32 = jnp.float32
I32 = jnp.int32


def _ru(x, m):
    return (x + m - 1) // m * m


def _mm_body(x_ref, w_ref, b_ref, o_ref, *, relu):
    y = jnp.dot(x_ref[...], w_ref[...], preferred_element_type=F32) + b_ref[...]
    if relu:
        y = jnp.maximum(y, 0.0)
    o_ref[...] = y


def _row_tile(n):
    for t in (2048, 1024, 1000, 512, 500, 256, 200, 128, 8):
        if n % t == 0:
            return t
    return n


def _mm(x, w, b, relu):
    n, k = x.shape
    m = w.shape[1]
    tr = _row_tile(n)
    return pl.pallas_call(
        functools.partial(_mm_body, relu=relu),
        grid=(n // tr,),
        in_specs=[
            pl.BlockSpec((tr, k), lambda i: (i, 0)),
            pl.BlockSpec((k, m), lambda i: (0, 0)),
            pl.BlockSpec((1, m), lambda i: (0, 0)),
        ],
        out_specs=pl.BlockSpec((tr, m), lambda i: (i, 0)),
        out_shape=jax.ShapeDtypeStruct((n, m), F32),
    )(x, w, b)


def _nodeupd_body(x_ref, nb_ref, w1_ref, w2_ref, b_ref, o_ref):
    nb = nb_ref[0] + nb_ref[1]
    y = (
        jnp.dot(x_ref[...], w1_ref[...], preferred_element_type=F32)
        + jnp.dot(nb, w2_ref[...], preferred_element_type=F32)
        + b_ref[...]
    )
    o_ref[...] = jnp.maximum(y, 0.0)


def _nodeupd(x, nbr, w1, w2, b):
    n, k = x.shape
    m = w1.shape[1]
    tr = _row_tile(n)
    return pl.pallas_call(
        _nodeupd_body,
        grid=(n // tr,),
        in_specs=[
            pl.BlockSpec((tr, k), lambda i: (i, 0)),
            pl.BlockSpec((2, tr, k), lambda i: (0, i, 0)),
            pl.BlockSpec((k, m), lambda i: (0, 0)),
            pl.BlockSpec((k, m), lambda i: (0, 0)),
            pl.BlockSpec((1, m), lambda i: (0, 0)),
        ],
        out_specs=pl.BlockSpec((tr, m), lambda i: (i, 0)),
        out_shape=jax.ShapeDtypeStruct((n, m), F32),
    )(x, nbr, w1, w2, b)


def _remap_body(d_ref, o_ref, *, vb):
    k = pl.program_id(0)
    lo = k * vb
    d = d_ref[...]
    m = (d >= lo) & (d < lo + vb)
    o_ref[...] = jnp.where(m, d - lo, vb)


def _remap(dst2, nb, vb):
    r = dst2.shape[0]
    tr = _row_tile(r)
    return pl.pallas_call(
        functools.partial(_remap_body, vb=vb),
        grid=(nb, r // tr),
        in_specs=[pl.BlockSpec((tr, 128), lambda k, i: (i, 0))],
        out_specs=pl.BlockSpec((tr, 128), lambda k, i, rt=r // tr: (k * rt + i, 0)),
        out_shape=jax.ShapeDtypeStruct((nb * r, 128), I32),
    )(dst2)


def _diff_body(p_ref, r_ref, o_ref):
    o_ref[...] = p_ref[...] - r_ref[...]


def _diff(p_h, r_h, nc):
    v, f = r_h.shape
    tr = _row_tile(v)
    return pl.pallas_call(
        _diff_body,
        grid=(nc, v // tr),
        in_specs=[
            pl.BlockSpec((tr, f), lambda c, i, vt=v // tr: (c * vt + i, 0)),
            pl.BlockSpec((tr, f), lambda c, i: (i, 0)),
        ],
        out_specs=pl.BlockSpec((tr, f), lambda c, i, vt=v // tr: (c * vt + i, 0)),
        out_shape=jax.ShapeDtypeStruct((nc * v, f), F32),
    )(p_h, r_h)


def _pool_body(x_ref, o_ref):
    part = jnp.sum(x_ref[...], axis=0, keepdims=True)
    c = pl.program_id(0)

    @pl.when((c == 0) & (pl.program_id(1) == 0))
    def _():
        o_ref[...] = jnp.zeros_like(o_ref)

    rows = lax.broadcasted_iota(I32, (8, 1), 0)
    o_ref[...] = o_ref[...] + jnp.where(rows == c, part, 0.0)


def _pool(x, nc, v):
    f = x.shape[1]
    tr = _row_tile(v)
    out = pl.pallas_call(
        _pool_body,
        grid=(nc, v // tr),
        in_specs=[pl.BlockSpec((tr, f), lambda c, i, vt=v // tr: (c * vt + i, 0))],
        out_specs=pl.BlockSpec((8, f), lambda c, i: (0, 0)),
        out_shape=jax.ShapeDtypeStruct((8, f), F32),
    )(x)
    return out[:nc]


def _mlp_body(g_ref, w1_ref, b1_ref, w2_ref, b2_ref, sc_ref, o_ref):
    h = jnp.maximum(
        jnp.dot(g_ref[...], w1_ref[...], preferred_element_type=F32) + b1_ref[...],
        0.0,
    )
    o_ref[...] = (
        jnp.sum(h * w2_ref[...], axis=1, keepdims=True) + b2_ref[...] + sc_ref[...]
    )


def _mlp(g, w1, b1, w2row, b2, scores):
    nc = g.shape[0]
    return pl.pallas_call(
        _mlp_body,
        out_shape=jax.ShapeDtypeStruct((nc, 1), F32),
    )(g, w1, b1, w2row, b2, scores)


_NSUB = 16
_BLK = 128
_MAXACC = 11776


def _graph_cfg(v, e):
    ep = _ru(e, 32 * 1024)
    nb = -(-v // _MAXACC)
    vb = _ru(-(-v // nb), _BLK)
    capt = ep // 32
    return ep, nb, vb, capt


@functools.lru_cache(maxsize=None)
def _make_edge_op(v, e):
    ep, nb, vb, capt = _graph_cfg(v, e)
    ra = vb + _BLK
    vo = nb * vb
    cpr = capt // _BLK
    rr = ep // _BLK
    ngrp = capt // (8 * _BLK)
    zrows = ra // _NSUB
    rps = vb // _NSUB
    mesh = plsc.VectorSubcoreMesh(core_axis_name="c", subcore_axis_name="s")

    @functools.partial(
        pl.kernel,
        out_type=jax.ShapeDtypeStruct((2, vo, _BLK), F32),
        mesh=mesh,
        scratch_types=[
            pltpu.VMEM_SHARED((ra, _BLK), F32),
            pltpu.VMEM((8, _BLK), I32),
            pltpu.VMEM((8, _BLK), I32),
            [pltpu.VMEM((_BLK, _BLK), F32)] * 2,
            pltpu.SemaphoreType.DMA,
            [pltpu.SemaphoreType.DMA] * 2,
            pltpu.SemaphoreType.DMA,
        ],
    )
    def edge_op(p_hbm, q_hbm, src2, didx_all, out, acc, sidx, didx, qbufs,
                qsem, gsems, ssem):
        c = lax.axis_index("c")
        s = lax.axis_index("s")
        t = c * _NSUB + s

        for k in range(nb):
            @pl.loop(0, _BLK)
            def _(r):
                for kk in range(8):
                    qbufs[0][r, pl.ds(kk * 16, 16)] = jnp.zeros((16,), F32)

            zb0 = pl.multiple_of(s * zrows, 8)
            off = 0
            while off < zrows:
                blk = min(_BLK, zrows - off)
                pltpu.sync_copy(qbufs[0].at[pl.ds(0, blk)],
                                acc.at[pl.ds(zb0 + off, blk)])
                off += blk
            plsc.subcore_barrier()

            @pl.loop(0, ngrp)
            def _(g):
                rb = pl.multiple_of(t * cpr + g * 8, 8)
                pltpu.sync_copy(src2.at[pl.ds(rb, 8)], sidx)
                pltpu.sync_copy(didx_all.at[pl.ds(k * rr + rb, 8)], didx)
                for w in range(4):
                    qds = []
                    for b in range(2):
                        j = w * 2 + b
                        eoff = pl.multiple_of((rb + j) * _BLK, _BLK)
                        qds.append(pltpu.async_copy(
                            q_hbm.at[pl.ds(eoff, _BLK)], qbufs[b], qsem))
                    for d in qds:
                        d.wait()
                    gds = [
                        pltpu.async_copy(
                            p_hbm.at[sidx.at[w * 2 + b]], qbufs[b], gsems[b],
                            add=True)
                        for b in range(2)
                    ]
                    sds = []
                    for b in range(2):
                        gds[b].wait()

                        @pl.loop(0, _BLK, unroll=4)
                        def _(r, _b=b):
                            for kk in range(8):
                                sl = pl.ds(kk * 16, 16)
                                qbufs[_b][r, sl] = jnp.maximum(qbufs[_b][r, sl], 0.0)

                        sds.append(pltpu.async_copy(
                            qbufs[b], acc.at[didx.at[w * 2 + b]], ssem, add=True))
                    for d in sds:
                        d.wait()

            plsc.subcore_barrier()
            r0 = pl.multiple_of(s * rps, 8)
            pltpu.sync_copy(
                acc.at[pl.ds(r0, rps)],
                out.at[c, pl.ds(pl.multiple_of(k * vb + r0, 8), rps)],
            )
            plsc.subcore_barrier()

    return edge_op


def _pad_graph(edge_index, edge_feats, v, ep):
    e = edge_feats.shape[0]
    src = edge_index[0].astype(I32)
    dst = edge_index[1].astype(I32)
    src = jnp.pad(src, (0, ep - e)).reshape(ep // _BLK, _BLK)
    dst = jnp.pad(dst, (0, ep - e), constant_values=v).reshape(ep // _BLK, _BLK)
    ef = jnp.pad(edge_feats, ((0, ep - e), (0, 0)))
    return src, dst, ef


def kernel(reactant_edge_index, reactant_node_feats, reactant_edge_feats,
           product_edge_index, product_node_feats, product_edge_feats,
           candidate_scores, gnn_Wp, gnn_bp, gnn_Wm, gnn_bm, gnn_Wn, gnn_bn,
           dgnn_Wm, dgnn_bm, dgnn_Wn, dgnn_bn, pred_W1, pred_b1, pred_W2,
           pred_b2):
    v, f = reactant_node_feats.shape
    vp = product_node_feats.shape[0]
    nc = candidate_scores.shape[0]
    er = reactant_edge_feats.shape[0]
    ep_e = product_edge_feats.shape[0]

    epr, nbr_r, vbr, _ = _graph_cfg(v, er)
    epp, nbr_p, vbp, _ = _graph_cfg(vp, ep_e)

    r_src, r_dst, r_ef = _pad_graph(reactant_edge_index, reactant_edge_feats, v, epr)
    p_src, p_dst, p_ef = _pad_graph(product_edge_index, product_edge_feats, vp, epp)

    bp_ = gnn_bp.reshape(1, -1)
    bm_ = gnn_bm.reshape(1, -1)
    bn_ = gnn_bn.reshape(1, -1)
    dbm_ = dgnn_bm.reshape(1, -1)
    dbn_ = dgnn_bn.reshape(1, -1)
    wm_h, wm_e = gnn_Wm[:f], gnn_Wm[f:]
    wn_1, wn_2 = gnn_Wn[:f], gnn_Wn[f:]
    dwm_h, dwm_e = dgnn_Wm[:f], dgnn_Wm[f:]
    dwn_1, dwn_2 = dgnn_Wn[:f], dgnn_Wn[f:]
    zb = jnp.zeros((1, f), F32)

    rdix = _remap(r_dst, nbr_r, vbr)
    pdix = _remap(p_dst, nbr_p, vbp)
    edge_r = _make_edge_op(v, er)
    edge_p = _make_edge_op(vp, ep_e)

    q_r = _mm(r_ef, wm_e, bm_, relu=False)
    q_p = _mm(p_ef, wm_e, bm_, relu=False)
    q_d = _mm(p_ef, dwm_e, dbm_, relu=False)

    r_h = _mm(reactant_node_feats, gnn_Wp, bp_, relu=True)
    p_h = _mm(product_node_feats, gnn_Wp, bp_, relu=True)

    for _ in range(3):
        pr = _mm(r_h, wm_h, zb, relu=False)
        nbr = edge_r(pr, q_r, r_src, rdix)
        r_h = _nodeupd(r_h, nbr, wn_1, wn_2, bn_)

        pp = _mm(p_h, wm_h, zb, relu=False)
        nbrp = edge_p(pp, q_p, p_src, pdix)
        p_h = _nodeupd(p_h, nbrp, wn_1, wn_2, bn_)

    d_h = _diff(p_h, r_h, nc)
    pd = _mm(d_h, dwm_h, zb, relu=False)
    nbrd = edge_p(pd, q_d, p_src, pdix)
    d_h = _nodeupd(d_h, nbrd, dwn_1, dwn_2, dbn_)

    g_feats = _pool(d_h, nc, v)
    return _mlp(g_feats, pred_W1, pred_b1.reshape(1, -1),
                pred_W2.reshape(1, -1), pred_b2.reshape(1, 1), candidate_scores)


# speedup vs baseline: 3.2662x; 3.0456x over previous
import functools

import jax
import jax.numpy as jnp
from jax import lax
from jax.experimental import pallas as pl
from jax.experimental.pallas import tpu as pltpu
from jax.experimental.pallas import tpu_sc as plsc

F32 = jnp.float32
I32 = jnp.int32


def _ru(x, m):
    return (x + m - 1) // m * m


def _mm_body(x_ref, w_ref, b_ref, o_ref, *, relu):
    y = jnp.dot(x_ref[...], w_ref[...], preferred_element_type=F32) + b_ref[...]
    if relu:
        y = jnp.maximum(y, 0.0)
    o_ref[...] = y


def _row_tile(n):
    for t in (2048, 1024, 1000, 512, 500, 256, 200, 128, 8):
        if n % t == 0:
            return t
    return n


def _mm(x, w, b, relu):
    n, k = x.shape
    m = w.shape[1]
    tr = _row_tile(n)
    return pl.pallas_call(
        functools.partial(_mm_body, relu=relu),
        grid=(n // tr,),
        in_specs=[
            pl.BlockSpec((tr, k), lambda i: (i, 0)),
            pl.BlockSpec((k, m), lambda i: (0, 0)),
            pl.BlockSpec((1, m), lambda i: (0, 0)),
        ],
        out_specs=pl.BlockSpec((tr, m), lambda i: (i, 0)),
        out_shape=jax.ShapeDtypeStruct((n, m), F32),
    )(x, w, b)


def _nodeupd_body(x_ref, nb_ref, w1_ref, w2_ref, b_ref, o_ref):
    nb = nb_ref[0] + nb_ref[1]
    y = (
        jnp.dot(x_ref[...], w1_ref[...], preferred_element_type=F32)
        + jnp.dot(nb, w2_ref[...], preferred_element_type=F32)
        + b_ref[...]
    )
    o_ref[...] = jnp.maximum(y, 0.0)


def _nodeupd(x, nbr, w1, w2, b):
    n, k = x.shape
    m = w1.shape[1]
    tr = _row_tile(n)
    return pl.pallas_call(
        _nodeupd_body,
        grid=(n // tr,),
        in_specs=[
            pl.BlockSpec((tr, k), lambda i: (i, 0)),
            pl.BlockSpec((2, tr, k), lambda i: (0, i, 0)),
            pl.BlockSpec((k, m), lambda i: (0, 0)),
            pl.BlockSpec((k, m), lambda i: (0, 0)),
            pl.BlockSpec((1, m), lambda i: (0, 0)),
        ],
        out_specs=pl.BlockSpec((tr, m), lambda i: (i, 0)),
        out_shape=jax.ShapeDtypeStruct((n, m), F32),
    )(x, nbr, w1, w2, b)


def _remap_body(d_ref, o_ref, *, vb):
    k = pl.program_id(0)
    lo = k * vb
    d = d_ref[...]
    m = (d >= lo) & (d < lo + vb)
    o_ref[...] = jnp.where(m, d - lo, vb)


def _remap(dst2, nb, vb):
    r = dst2.shape[0]
    tr = _row_tile(r)
    return pl.pallas_call(
        functools.partial(_remap_body, vb=vb),
        grid=(nb, r // tr),
        in_specs=[pl.BlockSpec((tr, 128), lambda k, i: (i, 0))],
        out_specs=pl.BlockSpec((tr, 128), lambda k, i, rt=r // tr: (k * rt + i, 0)),
        out_shape=jax.ShapeDtypeStruct((nb * r, 128), I32),
    )(dst2)


def _diff_body(p_ref, r_ref, o_ref):
    o_ref[...] = p_ref[...] - r_ref[...]


def _diff(p_h, r_h, nc):
    v, f = r_h.shape
    tr = _row_tile(v)
    return pl.pallas_call(
        _diff_body,
        grid=(nc, v // tr),
        in_specs=[
            pl.BlockSpec((tr, f), lambda c, i, vt=v // tr: (c * vt + i, 0)),
            pl.BlockSpec((tr, f), lambda c, i: (i, 0)),
        ],
        out_specs=pl.BlockSpec((tr, f), lambda c, i, vt=v // tr: (c * vt + i, 0)),
        out_shape=jax.ShapeDtypeStruct((nc * v, f), F32),
    )(p_h, r_h)


def _pool_body(x_ref, o_ref):
    part = jnp.sum(x_ref[...], axis=0, keepdims=True)
    c = pl.program_id(0)

    @pl.when((c == 0) & (pl.program_id(1) == 0))
    def _():
        o_ref[...] = jnp.zeros_like(o_ref)

    rows = lax.broadcasted_iota(I32, (8, 1), 0)
    o_ref[...] = o_ref[...] + jnp.where(rows == c, part, 0.0)


def _pool(x, nc, v):
    f = x.shape[1]
    tr = _row_tile(v)
    out = pl.pallas_call(
        _pool_body,
        grid=(nc, v // tr),
        in_specs=[pl.BlockSpec((tr, f), lambda c, i, vt=v // tr: (c * vt + i, 0))],
        out_specs=pl.BlockSpec((8, f), lambda c, i: (0, 0)),
        out_shape=jax.ShapeDtypeStruct((8, f), F32),
    )(x)
    return out[:nc]


def _mlp_body(g_ref, w1_ref, b1_ref, w2_ref, b2_ref, sc_ref, o_ref):
    h = jnp.maximum(
        jnp.dot(g_ref[...], w1_ref[...], preferred_element_type=F32) + b1_ref[...],
        0.0,
    )
    o_ref[...] = (
        jnp.sum(h * w2_ref[...], axis=1, keepdims=True) + b2_ref[...] + sc_ref[...]
    )


def _mlp(g, w1, b1, w2row, b2, scores):
    nc = g.shape[0]
    return pl.pallas_call(
        _mlp_body,
        out_shape=jax.ShapeDtypeStruct((nc, 1), F32),
    )(g, w1, b1, w2row, b2, scores)


_NSUB = 16
_BLK = 128
_MAXACC = 11776


def _graph_cfg(v, e):
    ep = _ru(e, 32 * 1024)
    nb = -(-v // _MAXACC)
    vb = _ru(-(-v // nb), _BLK)
    capt = ep // 32
    return ep, nb, vb, capt


@functools.lru_cache(maxsize=None)
def _make_edge_op(v, e):
    ep, nb, vb, capt = _graph_cfg(v, e)
    ra = vb + _BLK
    vo = nb * vb
    cpr = capt // _BLK
    rr = ep // _BLK
    ngrp = capt // (8 * _BLK)
    zrows = ra // _NSUB
    rps = vb // _NSUB
    mesh = plsc.VectorSubcoreMesh(core_axis_name="c", subcore_axis_name="s")

    @functools.partial(
        pl.kernel,
        out_type=jax.ShapeDtypeStruct((2, vo, _BLK), F32),
        mesh=mesh,
        scratch_types=[
            pltpu.VMEM_SHARED((ra, _BLK), F32),
            pltpu.VMEM((8, _BLK), I32),
            pltpu.VMEM((8, _BLK), I32),
            [pltpu.VMEM((_BLK, _BLK), F32)] * 2,
            pltpu.VMEM((8, _BLK), I32),
            pltpu.SemaphoreType.DMA,
            [pltpu.SemaphoreType.DMA] * 2,
            pltpu.SemaphoreType.DMA,
        ],
    )
    def edge_op(p_hbm, q_hbm, src2, didx_all, bounds, out, acc, sidx, didx,
                qbufs, cbuf, qsem, gsems, ssem):
        c = lax.axis_index("c")
        s = lax.axis_index("s")
        t = c * _NSUB + s
        pltpu.sync_copy(bounds, cbuf)

        for k in range(nb):
            @pl.loop(0, _BLK)
            def _(r):
                for kk in range(8):
                    qbufs[0][r, pl.ds(kk * 16, 16)] = jnp.zeros((16,), F32)

            zb0 = pl.multiple_of(s * zrows, 8)
            off = 0
            while off < zrows:
                blk = min(_BLK, zrows - off)
                pltpu.sync_copy(qbufs[0].at[pl.ds(0, blk)],
                                acc.at[pl.ds(zb0 + off, blk)])
                off += blk
            plsc.subcore_barrier()

            g0 = cbuf[0, pl.ds(0, 16)][k]
            g1 = cbuf[1, pl.ds(0, 16)][k]
            nt = jnp.maximum(g1 - g0 - t + 31, 0) // 32

            @pl.loop(0, nt)
            def _(i):
                g = g0 + t + i * 32
                rb = pl.multiple_of(g * 8, 8)
                pltpu.sync_copy(src2.at[pl.ds(rb, 8)], sidx)
                pltpu.sync_copy(didx_all.at[pl.ds(k * rr + rb, 8)], didx)
                for w in range(4):
                    qds = []
                    for b in range(2):
                        j = w * 2 + b
                        eoff = pl.multiple_of((rb + j) * _BLK, _BLK)
                        qds.append(pltpu.async_copy(
                            q_hbm.at[pl.ds(eoff, _BLK)], qbufs[b], qsem))
                    for d in qds:
                        d.wait()
                    gds = [
                        pltpu.async_copy(
                            p_hbm.at[sidx.at[w * 2 + b]], qbufs[b], gsems[b],
                            add=True)
                        for b in range(2)
                    ]
                    sds = []
                    for b in range(2):
                        gds[b].wait()

                        @pl.loop(0, _BLK, unroll=4)
                        def _(r, _b=b):
                            for kk in range(8):
                                sl = pl.ds(kk * 16, 16)
                                qbufs[_b][r, sl] = jnp.maximum(qbufs[_b][r, sl], 0.0)

                        sds.append(pltpu.async_copy(
                            qbufs[b], acc.at[didx.at[w * 2 + b]], ssem, add=True))
                    for d in sds:
                        d.wait()

            plsc.subcore_barrier()
            r0 = pl.multiple_of(s * rps, 8)
            pltpu.sync_copy(
                acc.at[pl.ds(r0, rps)],
                out.at[c, pl.ds(pl.multiple_of(k * vb + r0, 8), rps)],
            )
            plsc.subcore_barrier()

    return edge_op


def _pad_graph(edge_index, edge_feats, v, ep, nb, vb):
    e = edge_feats.shape[0]
    src = jnp.pad(edge_index[0].astype(I32), (0, ep - e))
    dst = jnp.pad(edge_index[1].astype(I32), (0, ep - e), constant_values=v)
    ef = jnp.pad(edge_feats, ((0, ep - e), (0, 0)))
    perm = jnp.argsort(dst)
    src = src[perm].reshape(ep // _BLK, _BLK)
    dsts = dst[perm]
    ef = ef[perm]
    edges = jnp.searchsorted(dsts, jnp.arange(nb + 1, dtype=I32) * vb).astype(I32)
    g0 = edges[:-1] // 1024
    g1 = (edges[1:] + 1023) // 1024
    bounds = jnp.zeros((8, _BLK), I32).at[0, :nb].set(g0).at[1, :nb].set(g1)
    return src, dsts.reshape(ep // _BLK, _BLK), ef, bounds


def kernel(reactant_edge_index, reactant_node_feats, reactant_edge_feats,
           product_edge_index, product_node_feats, product_edge_feats,
           candidate_scores, gnn_Wp, gnn_bp, gnn_Wm, gnn_bm, gnn_Wn, gnn_bn,
           dgnn_Wm, dgnn_bm, dgnn_Wn, dgnn_bn, pred_W1, pred_b1, pred_W2,
           pred_b2):
    v, f = reactant_node_feats.shape
    vp = product_node_feats.shape[0]
    nc = candidate_scores.shape[0]
    er = reactant_edge_feats.shape[0]
    ep_e = product_edge_feats.shape[0]

    epr, nbr_r, vbr, _ = _graph_cfg(v, er)
    epp, nbr_p, vbp, _ = _graph_cfg(vp, ep_e)

    r_src, r_dst, r_ef, r_bnd = _pad_graph(
        reactant_edge_index, reactant_edge_feats, v, epr, nbr_r, vbr)
    p_src, p_dst, p_ef, p_bnd = _pad_graph(
        product_edge_index, product_edge_feats, vp, epp, nbr_p, vbp)

    bp_ = gnn_bp.reshape(1, -1)
    bm_ = gnn_bm.reshape(1, -1)
    bn_ = gnn_bn.reshape(1, -1)
    dbm_ = dgnn_bm.reshape(1, -1)
    dbn_ = dgnn_bn.reshape(1, -1)
    wm_h, wm_e = gnn_Wm[:f], gnn_Wm[f:]
    wn_1, wn_2 = gnn_Wn[:f], gnn_Wn[f:]
    dwm_h, dwm_e = dgnn_Wm[:f], dgnn_Wm[f:]
    dwn_1, dwn_2 = dgnn_Wn[:f], dgnn_Wn[f:]
    zb = jnp.zeros((1, f), F32)

    rdix = _remap(r_dst, nbr_r, vbr)
    pdix = _remap(p_dst, nbr_p, vbp)
    edge_r = _make_edge_op(v, er)
    edge_p = _make_edge_op(vp, ep_e)

    q_r = _mm(r_ef, wm_e, bm_, relu=False)
    q_p = _mm(p_ef, wm_e, bm_, relu=False)
    q_d = _mm(p_ef, dwm_e, dbm_, relu=False)

    r_h = _mm(reactant_node_feats, gnn_Wp, bp_, relu=True)
    p_h = _mm(product_node_feats, gnn_Wp, bp_, relu=True)

    for _ in range(3):
        pr = _mm(r_h, wm_h, zb, relu=False)
        nbr = edge_r(pr, q_r, r_src, rdix, r_bnd)
        r_h = _nodeupd(r_h, nbr, wn_1, wn_2, bn_)

        pp = _mm(p_h, wm_h, zb, relu=False)
        nbrp = edge_p(pp, q_p, p_src, pdix, p_bnd)
        p_h = _nodeupd(p_h, nbrp, wn_1, wn_2, bn_)

    d_h = _diff(p_h, r_h, nc)
    pd = _mm(d_h, dwm_h, zb, relu=False)
    nbrd = edge_p(pd, q_d, p_src, pdix, p_bnd)
    d_h = _nodeupd(d_h, nbrd, dwn_1, dwn_2, dbn_)

    g_feats = _pool(d_h, nc, v)
    return _mlp(g_feats, pred_W1, pred_b1.reshape(1, -1),
                pred_W2.reshape(1, -1), pred_b2.reshape(1, 1), candidate_scores)
